# Initial kernel scaffold; baseline (speedup 1.0000x reference)
import jax
import jax.numpy as jnp
from jax.experimental import pallas as pl


def kernel(xyz, features, geo_w1, geo_b1, geo_g1, geo_be1, geo_w2, geo_b2, diff_w1, diff_b1, diff_g1, diff_be1, diff_w2, diff_b2, ep_w1, ep_b1, ep_w2, ep_b2, rf_w, rf_b, rf_g, rf_be):
    raise NotImplementedError("write your pallas kernel here")


# speedup vs baseline: 26.3663x; 26.3663x over previous
import functools

import jax
import jax.numpy as jnp
from jax import lax
from jax.experimental import pallas as pl
from jax.experimental.pallas import tpu as pltpu
from jax.experimental.pallas import tpu_sc as plsc

B, N, C, K = 8, 2048, 128, 16
H = C // 2
TCOLS = 128
BLK_A = 256
BLK_C = 128
NEG_INF = float("-inf")


def _topk_table_kernel(xyzT_ref, xyz_ref, feat_ref, w1_ref, gidx_ref, t_ref):
    b = pl.program_id(0)
    xb = xyz_ref[0]
    xT = xyzT_ref[0]
    fb = feat_ref[0]
    w1 = w1_ref[...]

    xx_all = jnp.sum(xT * xT, axis=0, keepdims=True)
    xx_blk = jnp.sum(xb * xb, axis=1, keepdims=True)

    xbl = xb.astype(jnp.bfloat16).astype(jnp.float32)
    xTl = xT.astype(jnp.bfloat16).astype(jnp.float32)
    mm = xbl[:, 0:1] * xTl[0:1, :]
    mm = mm + xbl[:, 1:2] * xTl[1:2, :]
    mm = mm + xbl[:, 2:3] * xTl[2:3, :]
    inner = -2.0 * mm
    d = (-xx_blk) - inner - xx_all

    lanes = lax.broadcasted_iota(jnp.int32, (BLK_A, N), 1)
    big = jnp.int32(N)
    for k in range(K):
        m = jnp.max(d, axis=1, keepdims=True)
        t = jnp.where(d == m, lanes, big)
        idxk = jnp.min(t, axis=1, keepdims=True)
        d = jnp.where(t == idxk, NEG_INF, d)
        gidx_ref[0, :, k] = (idxk[:, 0] + b * N).astype(jnp.int32)

    t_ref[0, :, 0:3] = xb
    t_ref[0, :, 3:64] = jnp.zeros((BLK_A, 61), jnp.float32)
    t_ref[0, :, 64:TCOLS] = jnp.dot(fb, w1, preferred_element_type=jnp.float32)


def _run_topk_table(xyz, xyzT, features, diff_w1):
    grid = (B, N // BLK_A)
    gidx, table = pl.pallas_call(
        _topk_table_kernel,
        grid=grid,
        in_specs=[
            pl.BlockSpec((1, 3, N), lambda b, i: (b, 0, 0)),
            pl.BlockSpec((1, BLK_A, 3), lambda b, i: (b, i, 0)),
            pl.BlockSpec((1, BLK_A, C), lambda b, i: (b, i, 0)),
            pl.BlockSpec((C, H), lambda b, i: (0, 0)),
        ],
        out_specs=[
            pl.BlockSpec((1, BLK_A, K), lambda b, i: (b, i, 0)),
            pl.BlockSpec((1, BLK_A, TCOLS), lambda b, i: (b, i, 0)),
        ],
        out_shape=[
            jax.ShapeDtypeStruct((B, N, K), jnp.int32),
            jax.ShapeDtypeStruct((B, N, TCOLS), jnp.float32),
        ],
    )(xyzT, xyz, features, diff_w1)
    return gidx, table


def _run_gather(table_flat, idx_flat):
    info = plsc.get_sparse_core_info()
    nw = info.num_cores * info.num_subcores
    total = B * N * K
    per_w = total // nw
    chunk = 128
    nchunks = per_w // chunk
    mesh = plsc.VectorSubcoreMesh(core_axis_name="c", subcore_axis_name="s")

    @functools.partial(
        pl.kernel,
        mesh=mesh,
        out_type=jax.ShapeDtypeStruct((total, TCOLS), jnp.float32),
        scratch_types=[
            pltpu.VMEM((chunk,), jnp.int32),
            pltpu.VMEM((chunk, TCOLS), jnp.float32),
            pltpu.SemaphoreType.DMA,
        ],
    )
    def gather_k(tf_hbm, idx_hbm, out_hbm, idx_v, rows_v, sem):
        wid = lax.axis_index("s") * info.num_cores + lax.axis_index("c")
        base_w = wid * per_w

        def body(j, carry):
            base = base_w + j * chunk
            pltpu.sync_copy(idx_hbm.at[pl.ds(base, chunk)], idx_v)
            pltpu.async_copy(tf_hbm.at[idx_v], rows_v, sem).wait()
            pltpu.sync_copy(rows_v, out_hbm.at[pl.ds(base, chunk)])
            return carry

        lax.fori_loop(0, nchunks, body, 0)

    return gather_k(table_flat, idx_flat)


_JACOBI_SEQ = ((0, 2), (1, 2), (0, 1))
_JACOBI_SWEEPS = 8


def _eigh3(cov6):
    a = dict(cov6)
    one = jnp.ones_like(a[(0, 0)])
    zero = jnp.zeros_like(one)
    v = [[one, zero, zero], [zero, one, zero], [zero, zero, one]]

    def get(i, j):
        return a[(i, j)] if i <= j else a[(j, i)]

    def put(i, j, val):
        a[(i, j) if i <= j else (j, i)] = val

    for _ in range(_JACOBI_SWEEPS):
        for (p, q) in _JACOBI_SEQ:
            r = 3 - p - q
            app = get(p, p); aqq = get(q, q); apq = get(p, q)
            apr = get(p, r); aqr = get(q, r)
            tau = (aqq - app) / (2.0 * apq)
            t = jnp.sign(tau) / (jnp.abs(tau) + jnp.sqrt(1.0 + tau * tau))
            t = jnp.where(apq == 0.0, 0.0, t)
            c = 1.0 / jnp.sqrt(1.0 + t * t)
            s = t * c
            put(p, p, c * (c * app - s * apq) - s * (c * apq - s * aqq))
            put(q, q, s * (s * app + c * apq) + c * (s * apq + c * aqq))
            put(p, q, zero)
            put(p, r, c * apr - s * aqr)
            put(q, r, s * apr + c * aqr)
            vp = v[p]; vq = v[q]
            v[p] = [c * vp[i] - s * vq[i] for i in range(3)]
            v[q] = [s * vp[i] + c * vq[i] for i in range(3)]

    w = [get(0, 0), get(1, 1), get(2, 2)]

    def cswap(i, j):
        swap = w[j] < w[i]
        wi = jnp.where(swap, w[j], w[i]); wj = jnp.where(swap, w[i], w[j])
        w[i] = wi; w[j] = wj
        vi = [jnp.where(swap, v[j][k], v[i][k]) for k in range(3)]
        vj = [jnp.where(swap, v[i][k], v[j][k]) for k in range(3)]
        v[i] = vi; v[j] = vj

    cswap(0, 1); cswap(1, 2); cswap(0, 1)
    return v[0], v[2]


def _layer_norm(x, g, b, eps=1e-5):
    m = jnp.mean(x, axis=-1, keepdims=True)
    var = jnp.mean((x - m) ** 2, axis=-1, keepdims=True)
    return (x - m) / jnp.sqrt(var + eps) * g + b


def _net_kernel(g_ref, t_ref, xyz_ref, feat_ref,
                geo_w1_ref, geo_b1_ref, geo_g1_ref, geo_be1_ref,
                geo_w2_ref, geo_b2_ref,
                diff_b1_ref, diff_g1_ref, diff_be1_ref,
                diff_w2_ref, diff_b2_ref,
                ep_w1_ref, ep_b1_ref, ep_w2t_ref, ep_b2_ref,
                rf_w_ref, rf_b_ref, rf_g_ref, rf_be_ref,
                refined_ref, edge_ref):
    g3 = g_ref[...]
    xyz_i = xyz_ref[...]
    feat = feat_ref[...]
    p_i = t_ref[...][:, 64:TCOLS]

    rel = g3[:, :, 0:3] - xyz_i[:, None, :]
    rx = rel[:, :, 0]; ry = rel[:, :, 1]; rz = rel[:, :, 2]
    dsq = rx * rx + ry * ry + rz * rz
    dist = jnp.sqrt(dsq + 1e-12)
    mean_dist = jnp.mean(dist, axis=1, keepdims=True)
    inv_k = 1.0 / K
    bx = rx.astype(jnp.bfloat16).astype(jnp.float32)
    by = ry.astype(jnp.bfloat16).astype(jnp.float32)
    bz = rz.astype(jnp.bfloat16).astype(jnp.float32)
    cov6 = {
        (0, 0): jnp.sum(bx * bx, axis=1, keepdims=True) * inv_k,
        (0, 1): jnp.sum(bx * by, axis=1, keepdims=True) * inv_k,
        (0, 2): jnp.sum(bx * bz, axis=1, keepdims=True) * inv_k,
        (1, 1): jnp.sum(by * by, axis=1, keepdims=True) * inv_k,
        (1, 2): jnp.sum(by * bz, axis=1, keepdims=True) * inv_k,
        (2, 2): jnp.sum(bz * bz, axis=1, keepdims=True) * inv_k,
    }
    mean_rel = [jnp.mean(rx, axis=1, keepdims=True),
                jnp.mean(ry, axis=1, keepdims=True),
                jnp.mean(rz, axis=1, keepdims=True)]
    normal, curv = _eigh3(cov6)
    geo_feat = jnp.concatenate(normal + curv + mean_rel + [mean_dist], axis=1)
    g1 = jax.nn.relu(_layer_norm(
        jnp.dot(geo_feat, geo_w1_ref[...],
                preferred_element_type=jnp.float32) + geo_b1_ref[...],
        geo_g1_ref[...], geo_be1_ref[...]))
    geo_encoded = jnp.dot(g1, geo_w2_ref[...],
                          preferred_element_type=jnp.float32) + geo_b2_ref[...]

    pn = g3[:, :, 64:TCOLS]
    h1 = (p_i[:, None, :] - pn) + diff_b1_ref[...]
    h1 = jax.nn.relu(_layer_norm(h1, diff_g1_ref[...], diff_be1_ref[...]))
    h1f = h1.reshape(BLK_C * K, H)
    d2 = jnp.dot(h1f, diff_w2_ref[...],
                 preferred_element_type=jnp.float32) + diff_b2_ref[...]
    diff_encoded = jnp.max(d2.reshape(BLK_C, K, H), axis=1)

    combined = jnp.concatenate([geo_encoded, diff_encoded], axis=1)

    e = jax.nn.relu(jnp.dot(combined, ep_w1_ref[...],
                            preferred_element_type=jnp.float32) + ep_b1_ref[...])
    logit = jnp.sum(e * ep_w2t_ref[...], axis=1, keepdims=True) + ep_b2_ref[...]
    edge_prob = jax.nn.sigmoid(logit)

    fc = jnp.concatenate([feat, combined], axis=1)
    r = jax.nn.relu(_layer_norm(
        jnp.dot(fc, rf_w_ref[...],
                preferred_element_type=jnp.float32) + rf_b_ref[...],
        rf_g_ref[...], rf_be_ref[...]))
    refined_ref[...] = feat + r * edge_prob
    edge_ref[...] = edge_prob


def _run_net(g4, table_flat, xyz_flat, feat_flat, ws):
    grid = (B * N // BLK_C,)
    full = lambda shape: pl.BlockSpec(shape, lambda i: tuple(0 for _ in shape))
    refined, edge = pl.pallas_call(
        _net_kernel,
        grid=grid,
        in_specs=[
            pl.BlockSpec((BLK_C, K, TCOLS), lambda i: (i, 0, 0)),
            pl.BlockSpec((BLK_C, TCOLS), lambda i: (i, 0)),
            pl.BlockSpec((BLK_C, 3), lambda i: (i, 0)),
            pl.BlockSpec((BLK_C, C), lambda i: (i, 0)),
            full((10, H)), full((H,)), full((H,)), full((H,)),
            full((H, H)), full((H,)),
            full((H,)), full((H,)), full((H,)),
            full((H, H)), full((H,)),
            full((C, C // 4)), full((C // 4,)), full((1, C // 4)), full((1,)),
            full((2 * C, C)), full((C,)), full((C,)), full((C,)),
        ],
        out_specs=[
            pl.BlockSpec((BLK_C, C), lambda i: (i, 0)),
            pl.BlockSpec((BLK_C, 1), lambda i: (i, 0)),
        ],
        out_shape=[
            jax.ShapeDtypeStruct((B * N, C), jnp.float32),
            jax.ShapeDtypeStruct((B * N, 1), jnp.float32),
        ],
    )(g4, table_flat, xyz_flat, feat_flat, *ws)
    return refined, edge


def kernel(xyz, features, geo_w1, geo_b1, geo_g1, geo_be1, geo_w2, geo_b2,
           diff_w1, diff_b1, diff_g1, diff_be1, diff_w2, diff_b2,
           ep_w1, ep_b1, ep_w2, ep_b2, rf_w, rf_b, rf_g, rf_be):
    xyzT = jnp.swapaxes(xyz, 1, 2)
    gidx, table = _run_topk_table(xyz, xyzT, features, diff_w1)

    table_flat = table.reshape(B * N, TCOLS)
    idx_flat = gidx.reshape(B * N * K)
    g = _run_gather(table_flat, idx_flat)
    g4 = g.reshape(B * N, K, TCOLS)

    ws = (geo_w1, geo_b1, geo_g1, geo_be1, geo_w2, geo_b2,
          diff_b1, diff_g1, diff_be1, diff_w2, diff_b2,
          ep_w1, ep_b1, ep_w2.reshape(1, C // 4), ep_b2,
          rf_w, rf_b, rf_g, rf_be)
    refined, edge = _run_net(g4, table_flat, xyz.reshape(B * N, 3),
                             features.reshape(B * N, C), ws)
    return refined.reshape(B, N, C), edge.reshape(B, N, 1)


# speedup vs baseline: 27.0673x; 1.0266x over previous
import functools

import jax
import jax.numpy as jnp
from jax import lax
from jax.experimental import pallas as pl
from jax.experimental.pallas import tpu as pltpu
from jax.experimental.pallas import tpu_sc as plsc

B, N, C, K = 8, 2048, 128, 16
H = C // 2
TCOLS = 128
BLK_A = 256
BLK_C = 128
NEG_INF = float("-inf")


def _topk_table_kernel(xyzT_ref, xyz_ref, feat_ref, w1_ref, gidx_ref, t_ref):
    b = pl.program_id(0)
    xb = xyz_ref[0]
    xT = xyzT_ref[0]
    fb = feat_ref[0]
    w1 = w1_ref[...]

    xx_all = jnp.sum(xT * xT, axis=0, keepdims=True)
    xx_blk = jnp.sum(xb * xb, axis=1, keepdims=True)

    xbl = xb.astype(jnp.bfloat16).astype(jnp.float32)
    xTl = xT.astype(jnp.bfloat16).astype(jnp.float32)
    mm = xbl[:, 0:1] * xTl[0:1, :]
    mm = mm + xbl[:, 1:2] * xTl[1:2, :]
    mm = mm + xbl[:, 2:3] * xTl[2:3, :]
    inner = -2.0 * mm
    d = (-xx_blk) - inner - xx_all

    lanes = lax.broadcasted_iota(jnp.int32, (BLK_A, N), 1)
    big = jnp.int32(N)
    for k in range(K):
        m = jnp.max(d, axis=1, keepdims=True)
        t = jnp.where(d == m, lanes, big)
        idxk = jnp.min(t, axis=1, keepdims=True)
        d = jnp.where(t == idxk, NEG_INF, d)
        gidx_ref[0, :, k] = (idxk[:, 0] + b * N).astype(jnp.int32)

    t_ref[0, :, 0:3] = xb
    t_ref[0, :, 3:64] = jnp.zeros((BLK_A, 61), jnp.float32)
    t_ref[0, :, 64:TCOLS] = jnp.dot(fb, w1, preferred_element_type=jnp.float32)


def _run_topk_table(xyz, xyzT, features, diff_w1):
    grid = (B, N // BLK_A)
    gidx, table = pl.pallas_call(
        _topk_table_kernel,
        grid=grid,
        in_specs=[
            pl.BlockSpec((1, 3, N), lambda b, i: (b, 0, 0)),
            pl.BlockSpec((1, BLK_A, 3), lambda b, i: (b, i, 0)),
            pl.BlockSpec((1, BLK_A, C), lambda b, i: (b, i, 0)),
            pl.BlockSpec((C, H), lambda b, i: (0, 0)),
        ],
        out_specs=[
            pl.BlockSpec((1, BLK_A, K), lambda b, i: (b, i, 0)),
            pl.BlockSpec((1, BLK_A, TCOLS), lambda b, i: (b, i, 0)),
        ],
        out_shape=[
            jax.ShapeDtypeStruct((B, N, K), jnp.int32),
            jax.ShapeDtypeStruct((B, N, TCOLS), jnp.float32),
        ],
    )(xyzT, xyz, features, diff_w1)
    return gidx, table


def _run_gather(table_flat, idx_flat):
    info = plsc.get_sparse_core_info()
    nw = info.num_cores * info.num_subcores
    total = B * N * K
    per_w = total // nw
    chunk = 128
    nchunks = per_w // chunk
    mesh = plsc.VectorSubcoreMesh(core_axis_name="c", subcore_axis_name="s")

    kk = 4
    slab = chunk * kk
    nslabs = per_w // slab

    @functools.partial(
        pl.kernel,
        mesh=mesh,
        out_type=jax.ShapeDtypeStruct((total, TCOLS), jnp.float32),
        scratch_types=[
            pltpu.VMEM((slab,), jnp.int32),
            pltpu.VMEM((slab, TCOLS), jnp.float32),
            pltpu.SemaphoreType.DMA,
        ],
    )
    def gather_k(tf_hbm, idx_hbm, out_hbm, idx_v, rows_v, sem):
        wid = lax.axis_index("s") * info.num_cores + lax.axis_index("c")
        base_w = wid * per_w

        def body(j, carry):
            base = base_w + j * slab
            pltpu.sync_copy(idx_hbm.at[pl.ds(base, slab)], idx_v)
            cps = [pltpu.async_copy(
                       tf_hbm.at[idx_v.at[pl.ds(b * chunk, chunk)]],
                       rows_v.at[pl.ds(b * chunk, chunk)], sem)
                   for b in range(kk)]
            for cp in cps:
                cp.wait()
            pltpu.sync_copy(rows_v, out_hbm.at[pl.ds(base, slab)])
            return carry

        lax.fori_loop(0, nslabs, body, 0)

    return gather_k(table_flat, idx_flat)


_JACOBI_SEQ = ((0, 2), (1, 2), (0, 1))
_JACOBI_SWEEPS = 8


def _eigh3(cov6):
    a = dict(cov6)
    one = jnp.ones_like(a[(0, 0)])
    zero = jnp.zeros_like(one)
    v = [[one, zero, zero], [zero, one, zero], [zero, zero, one]]

    def get(i, j):
        return a[(i, j)] if i <= j else a[(j, i)]

    def put(i, j, val):
        a[(i, j) if i <= j else (j, i)] = val

    for _ in range(_JACOBI_SWEEPS):
        for (p, q) in _JACOBI_SEQ:
            r = 3 - p - q
            app = get(p, p); aqq = get(q, q); apq = get(p, q)
            apr = get(p, r); aqr = get(q, r)
            tau = (aqq - app) / (2.0 * apq)
            t = jnp.sign(tau) / (jnp.abs(tau) + jnp.sqrt(1.0 + tau * tau))
            t = jnp.where(apq == 0.0, 0.0, t)
            c = 1.0 / jnp.sqrt(1.0 + t * t)
            s = t * c
            put(p, p, c * (c * app - s * apq) - s * (c * apq - s * aqq))
            put(q, q, s * (s * app + c * apq) + c * (s * apq + c * aqq))
            put(p, q, zero)
            put(p, r, c * apr - s * aqr)
            put(q, r, s * apr + c * aqr)
            vp = v[p]; vq = v[q]
            v[p] = [c * vp[i] - s * vq[i] for i in range(3)]
            v[q] = [s * vp[i] + c * vq[i] for i in range(3)]

    w = [get(0, 0), get(1, 1), get(2, 2)]

    def cswap(i, j):
        swap = w[j] < w[i]
        wi = jnp.where(swap, w[j], w[i]); wj = jnp.where(swap, w[i], w[j])
        w[i] = wi; w[j] = wj
        vi = [jnp.where(swap, v[j][k], v[i][k]) for k in range(3)]
        vj = [jnp.where(swap, v[i][k], v[j][k]) for k in range(3)]
        v[i] = vi; v[j] = vj

    cswap(0, 1); cswap(1, 2); cswap(0, 1)
    return v[0], v[2]


def _layer_norm(x, g, b, eps=1e-5):
    m = jnp.mean(x, axis=-1, keepdims=True)
    var = jnp.mean((x - m) ** 2, axis=-1, keepdims=True)
    return (x - m) / jnp.sqrt(var + eps) * g + b


def _net_kernel(g_ref, t_ref, xyz_ref, feat_ref,
                geo_w1_ref, geo_b1_ref, geo_g1_ref, geo_be1_ref,
                geo_w2_ref, geo_b2_ref,
                diff_b1_ref, diff_g1_ref, diff_be1_ref,
                diff_w2_ref, diff_b2_ref,
                ep_w1_ref, ep_b1_ref, ep_w2t_ref, ep_b2_ref,
                rf_w_ref, rf_b_ref, rf_g_ref, rf_be_ref,
                refined_ref, edge_ref):
    g3 = g_ref[...]
    xyz_i = xyz_ref[...]
    feat = feat_ref[...]
    p_i = t_ref[...][:, 64:TCOLS]

    rel = g3[:, :, 0:3] - xyz_i[:, None, :]
    rx = rel[:, :, 0]; ry = rel[:, :, 1]; rz = rel[:, :, 2]
    dsq = rx * rx + ry * ry + rz * rz
    dist = jnp.sqrt(dsq + 1e-12)
    mean_dist = jnp.mean(dist, axis=1, keepdims=True)
    inv_k = 1.0 / K
    bx = rx.astype(jnp.bfloat16).astype(jnp.float32)
    by = ry.astype(jnp.bfloat16).astype(jnp.float32)
    bz = rz.astype(jnp.bfloat16).astype(jnp.float32)
    lm = lambda x: x.reshape(1, BLK_C)
    cov6 = {
        (0, 0): lm(jnp.sum(bx * bx, axis=1, keepdims=True) * inv_k),
        (0, 1): lm(jnp.sum(bx * by, axis=1, keepdims=True) * inv_k),
        (0, 2): lm(jnp.sum(bx * bz, axis=1, keepdims=True) * inv_k),
        (1, 1): lm(jnp.sum(by * by, axis=1, keepdims=True) * inv_k),
        (1, 2): lm(jnp.sum(by * bz, axis=1, keepdims=True) * inv_k),
        (2, 2): lm(jnp.sum(bz * bz, axis=1, keepdims=True) * inv_k),
    }
    mean_rel = [jnp.mean(rx, axis=1, keepdims=True),
                jnp.mean(ry, axis=1, keepdims=True),
                jnp.mean(rz, axis=1, keepdims=True)]
    normal_l, curv_l = _eigh3(cov6)
    normal = [x.reshape(BLK_C, 1) for x in normal_l]
    curv = [x.reshape(BLK_C, 1) for x in curv_l]
    geo_feat = jnp.concatenate(normal + curv + mean_rel + [mean_dist], axis=1)
    g1 = jax.nn.relu(_layer_norm(
        jnp.dot(geo_feat, geo_w1_ref[...],
                preferred_element_type=jnp.float32) + geo_b1_ref[...],
        geo_g1_ref[...], geo_be1_ref[...]))
    geo_encoded = jnp.dot(g1, geo_w2_ref[...],
                          preferred_element_type=jnp.float32) + geo_b2_ref[...]

    pn = g3[:, :, 64:TCOLS]
    h1 = (p_i[:, None, :] - pn) + diff_b1_ref[...]
    h1 = jax.nn.relu(_layer_norm(h1, diff_g1_ref[...], diff_be1_ref[...]))
    h1f = h1.reshape(BLK_C * K, H)
    d2 = jnp.dot(h1f, diff_w2_ref[...],
                 preferred_element_type=jnp.float32) + diff_b2_ref[...]
    diff_encoded = jnp.max(d2.reshape(BLK_C, K, H), axis=1)

    combined = jnp.concatenate([geo_encoded, diff_encoded], axis=1)

    e = jax.nn.relu(jnp.dot(combined, ep_w1_ref[...],
                            preferred_element_type=jnp.float32) + ep_b1_ref[...])
    logit = jnp.sum(e * ep_w2t_ref[...], axis=1, keepdims=True) + ep_b2_ref[...]
    edge_prob = jax.nn.sigmoid(logit)

    fc = jnp.concatenate([feat, combined], axis=1)
    r = jax.nn.relu(_layer_norm(
        jnp.dot(fc, rf_w_ref[...],
                preferred_element_type=jnp.float32) + rf_b_ref[...],
        rf_g_ref[...], rf_be_ref[...]))
    refined_ref[...] = feat + r * edge_prob
    edge_ref[...] = edge_prob


def _run_net(g4, table_flat, xyz_flat, feat_flat, ws):
    grid = (B * N // BLK_C,)
    full = lambda shape: pl.BlockSpec(shape, lambda i: tuple(0 for _ in shape))
    refined, edge = pl.pallas_call(
        _net_kernel,
        grid=grid,
        in_specs=[
            pl.BlockSpec((BLK_C, K, TCOLS), lambda i: (i, 0, 0)),
            pl.BlockSpec((BLK_C, TCOLS), lambda i: (i, 0)),
            pl.BlockSpec((BLK_C, 3), lambda i: (i, 0)),
            pl.BlockSpec((BLK_C, C), lambda i: (i, 0)),
            full((10, H)), full((H,)), full((H,)), full((H,)),
            full((H, H)), full((H,)),
            full((H,)), full((H,)), full((H,)),
            full((H, H)), full((H,)),
            full((C, C // 4)), full((C // 4,)), full((1, C // 4)), full((1,)),
            full((2 * C, C)), full((C,)), full((C,)), full((C,)),
        ],
        out_specs=[
            pl.BlockSpec((BLK_C, C), lambda i: (i, 0)),
            pl.BlockSpec((BLK_C, 1), lambda i: (i, 0)),
        ],
        out_shape=[
            jax.ShapeDtypeStruct((B * N, C), jnp.float32),
            jax.ShapeDtypeStruct((B * N, 1), jnp.float32),
        ],
    )(g4, table_flat, xyz_flat, feat_flat, *ws)
    return refined, edge


def kernel(xyz, features, geo_w1, geo_b1, geo_g1, geo_be1, geo_w2, geo_b2,
           diff_w1, diff_b1, diff_g1, diff_be1, diff_w2, diff_b2,
           ep_w1, ep_b1, ep_w2, ep_b2, rf_w, rf_b, rf_g, rf_be):
    xyzT = jnp.swapaxes(xyz, 1, 2)
    gidx, table = _run_topk_table(xyz, xyzT, features, diff_w1)

    table_flat = table.reshape(B * N, TCOLS)
    idx_flat = gidx.reshape(B * N * K)
    g = _run_gather(table_flat, idx_flat)
    g4 = g.reshape(B * N, K, TCOLS)

    ws = (geo_w1, geo_b1, geo_g1, geo_be1, geo_w2, geo_b2,
          diff_b1, diff_g1, diff_be1, diff_w2, diff_b2,
          ep_w1, ep_b1, ep_w2.reshape(1, C // 4), ep_b2,
          rf_w, rf_b, rf_g, rf_be)
    refined, edge = _run_net(g4, table_flat, xyz.reshape(B * N, 3),
                             features.reshape(B * N, C), ws)
    return refined.reshape(B, N, C), edge.reshape(B, N, 1)
